# baseline (speedup 1.0000x reference)
import jax
import jax.numpy as jnp
from jax.experimental import pallas as pl
from jax.experimental.pallas import tpu as pltpu

ROWS = 32
H = 512
W_DIM = 512
C = 32
NBLK = H // ROWS
ST = 640
MARGIN = 128
NLANES = ROWS * ST
XW = MARGIN + (ROWS + 2) * ST + 128
BLK = ROWS * W_DIM


def _row(r):
    return MARGIN + r * ST


def _conv_body(x_ref, w3_ref, b_ref, out_ref, xpad_ref):
    i = pl.program_id(1)

    @pl.when(i == 0)
    def _():
        xpad_ref[...] = jnp.zeros_like(xpad_ref)

    cur = x_ref[0]

    @pl.when(i > 0)
    def _():
        @pl.when(i < NBLK)
        def _():
            xpad_ref[:, _row(ROWS + 1):_row(ROWS + 1) + W_DIM] = cur[:, 0:W_DIM]

        @pl.when(i == NBLK)
        def _():
            xpad_ref[:, _row(ROWS + 1):_row(ROWS + 1) + W_DIM] = jnp.zeros(
                (C, W_DIM), jnp.float32)

        x0 = xpad_ref[...]
        xm = pltpu.roll(x0, 1, 1)
        xp = pltpu.roll(x0, XW - 1, 1)
        x0b = x0.astype(jnp.bfloat16)
        xmb = xm.astype(jnp.bfloat16)
        xpb = xp.astype(jnp.bfloat16)
        slabs = (xmb, x0b, xpb)
        xcol = jnp.concatenate(
            [slabs[kw][:, _row(kh):_row(kh) + NLANES]
             for kh in range(3) for kw in range(3)], axis=0)
        res = jax.lax.dot_general(
            w3_ref[...], xcol, (((1,), (0,)), ((), ())),
            preferred_element_type=jnp.float32)
        bias = b_ref[...]
        for r in range(ROWS):
            out_ref[0, :, r * W_DIM:(r + 1) * W_DIM] = (
                res[:, r * ST:r * ST + W_DIM] + bias)

    @pl.when(i < NBLK)
    def _():
        @pl.when(i > 0)
        def _():
            xpad_ref[:, _row(0):_row(0) + W_DIM] = (
                xpad_ref[:, _row(ROWS):_row(ROWS) + W_DIM])

        for r in range(ROWS):
            xpad_ref[:, _row(r + 1):_row(r + 1) + W_DIM] = (
                cur[:, r * W_DIM:(r + 1) * W_DIM])


def kernel(x, W, b):
    n = x.shape[0]
    xf = x.reshape(n, C, H * W_DIM)
    w3 = jnp.transpose(W, (0, 2, 3, 1)).reshape(C, 288).astype(jnp.bfloat16)
    bb = jnp.broadcast_to(b[:, None], (C, W_DIM))

    out = pl.pallas_call(
        _conv_body,
        grid=(n, NBLK + 1),
        in_specs=[
            pl.BlockSpec((1, C, BLK),
                         lambda nn, ii: (nn, 0, jnp.minimum(ii, NBLK - 1))),
            pl.BlockSpec((C, 288), lambda nn, ii: (0, 0)),
            pl.BlockSpec((C, W_DIM), lambda nn, ii: (0, 0)),
        ],
        out_specs=pl.BlockSpec((1, C, BLK),
                               lambda nn, ii: (nn, 0, jnp.maximum(ii - 1, 0))),
        out_shape=jax.ShapeDtypeStruct((n, C, H * W_DIM), jnp.float32),
        scratch_shapes=[pltpu.VMEM((C, XW), jnp.float32)],
        compiler_params=pltpu.CompilerParams(
            dimension_semantics=("parallel", "arbitrary"),
            vmem_limit_bytes=100 * 1024 * 1024,
        ),
    )(xf, w3, bb)
    return out.reshape(n, C, H, W_DIM)


# speedup vs baseline: 2.2703x; 2.2703x over previous
import jax
import jax.numpy as jnp
from jax.experimental import pallas as pl
from jax.experimental.pallas import tpu as pltpu

ROWS = 32
H = 512
W_DIM = 512
C = 32
NBLK = H // ROWS
ST = 640
MARGIN = 128
NLANES = ROWS * ST
XW = MARGIN + (ROWS + 2) * ST + 128


def _row(r):
    return MARGIN + r * ST


def _conv_body(x_ref, w3_ref, b_ref, out_ref, xpad_ref):
    i = pl.program_id(1)

    @pl.when(i == 0)
    def _():
        xpad_ref[...] = jnp.zeros_like(xpad_ref)

    xt = jnp.swapaxes(x_ref[0], 0, 1)

    @pl.when(i > 0)
    def _():
        @pl.when(i < NBLK)
        def _():
            xpad_ref[:, _row(ROWS + 1):_row(ROWS + 1) + W_DIM] = xt[0]

        @pl.when(i == NBLK)
        def _():
            xpad_ref[:, _row(ROWS + 1):_row(ROWS + 1) + W_DIM] = jnp.zeros(
                (C, W_DIM), jnp.float32)

        x0 = xpad_ref[...]
        xm = pltpu.roll(x0, 1, 1)
        xp = pltpu.roll(x0, XW - 1, 1)
        x0b = x0.astype(jnp.bfloat16)
        xmb = xm.astype(jnp.bfloat16)
        xpb = xp.astype(jnp.bfloat16)
        slabs = (xmb, x0b, xpb)
        xcol = jnp.concatenate(
            [slabs[kw][:, _row(kh):_row(kh) + NLANES]
             for kh in range(3) for kw in range(3)], axis=0)
        res = jax.lax.dot_general(
            w3_ref[...], xcol, (((1,), (0,)), ((), ())),
            preferred_element_type=jnp.float32)
        bias = b_ref[...]
        res3 = jnp.stack(
            [res[:, r * ST:r * ST + W_DIM] + bias for r in range(ROWS)],
            axis=0)
        out_ref[0] = jnp.swapaxes(res3, 0, 1)

    @pl.when(i < NBLK)
    def _():
        @pl.when(i > 0)
        def _():
            xpad_ref[:, _row(0):_row(0) + W_DIM] = (
                xpad_ref[:, _row(ROWS):_row(ROWS) + W_DIM])

        for r in range(ROWS):
            xpad_ref[:, _row(r + 1):_row(r + 1) + W_DIM] = xt[r]


def kernel(x, W, b):
    n = x.shape[0]
    w3 = jnp.transpose(W, (0, 2, 3, 1)).reshape(C, 288).astype(jnp.bfloat16)
    bb = jnp.broadcast_to(b[:, None], (C, W_DIM))

    return pl.pallas_call(
        _conv_body,
        grid=(n, NBLK + 1),
        in_specs=[
            pl.BlockSpec((1, C, ROWS, W_DIM),
                         lambda nn, ii: (nn, 0, jnp.minimum(ii, NBLK - 1), 0)),
            pl.BlockSpec((C, 288), lambda nn, ii: (0, 0)),
            pl.BlockSpec((C, W_DIM), lambda nn, ii: (0, 0)),
        ],
        out_specs=pl.BlockSpec((1, C, ROWS, W_DIM),
                               lambda nn, ii: (nn, 0, jnp.maximum(ii - 1, 0), 0)),
        out_shape=jax.ShapeDtypeStruct((n, C, H, W_DIM), jnp.float32),
        scratch_shapes=[pltpu.VMEM((C, XW), jnp.float32)],
        compiler_params=pltpu.CompilerParams(
            dimension_semantics=("parallel", "arbitrary"),
            vmem_limit_bytes=100 * 1024 * 1024,
        ),
    )(x, w3, bb)


# speedup vs baseline: 2.3155x; 1.0199x over previous
import jax
import jax.numpy as jnp
from jax.experimental import pallas as pl
from jax.experimental.pallas import tpu as pltpu

ROWS = 64
H = 512
W_DIM = 512
C = 32
NBLK = H // ROWS
ST = 640
MARGIN = 128
NLANES = ROWS * ST
XW = MARGIN + (ROWS + 2) * ST + 128


def _row(r):
    return MARGIN + r * ST


def _conv_body(x_ref, w3_ref, b_ref, out_ref, xpad_ref):
    i = pl.program_id(1)

    @pl.when(i == 0)
    def _():
        xpad_ref[...] = jnp.zeros_like(xpad_ref)

    xt = jnp.swapaxes(x_ref[0], 0, 1)

    @pl.when(i > 0)
    def _():
        @pl.when(i < NBLK)
        def _():
            xpad_ref[:, _row(ROWS + 1):_row(ROWS + 1) + W_DIM] = xt[0]

        @pl.when(i == NBLK)
        def _():
            xpad_ref[:, _row(ROWS + 1):_row(ROWS + 1) + W_DIM] = jnp.zeros(
                (C, W_DIM), jnp.float32)

        x0 = xpad_ref[...]
        xm = pltpu.roll(x0, 1, 1)
        xp = pltpu.roll(x0, XW - 1, 1)
        x0b = x0.astype(jnp.bfloat16)
        xmb = xm.astype(jnp.bfloat16)
        xpb = xp.astype(jnp.bfloat16)
        slabs = (xmb, x0b, xpb)
        xcol = jnp.concatenate(
            [slabs[kw][:, _row(kh):_row(kh) + NLANES]
             for kh in range(3) for kw in range(3)], axis=0)
        res = jax.lax.dot_general(
            w3_ref[...], xcol, (((1,), (0,)), ((), ())),
            preferred_element_type=jnp.float32)
        bias = b_ref[...]
        res3 = jnp.stack(
            [res[:, r * ST:r * ST + W_DIM] + bias for r in range(ROWS)],
            axis=0)
        out_ref[0] = jnp.swapaxes(res3, 0, 1)

    @pl.when(i < NBLK)
    def _():
        @pl.when(i > 0)
        def _():
            xpad_ref[:, _row(0):_row(0) + W_DIM] = (
                xpad_ref[:, _row(ROWS):_row(ROWS) + W_DIM])

        for r in range(ROWS):
            xpad_ref[:, _row(r + 1):_row(r + 1) + W_DIM] = xt[r]


def kernel(x, W, b):
    n = x.shape[0]
    w3 = jnp.transpose(W, (0, 2, 3, 1)).reshape(C, 288).astype(jnp.bfloat16)
    bb = jnp.broadcast_to(b[:, None], (C, W_DIM))

    return pl.pallas_call(
        _conv_body,
        grid=(n, NBLK + 1),
        in_specs=[
            pl.BlockSpec((1, C, ROWS, W_DIM),
                         lambda nn, ii: (nn, 0, jnp.minimum(ii, NBLK - 1), 0)),
            pl.BlockSpec((C, 288), lambda nn, ii: (0, 0)),
            pl.BlockSpec((C, W_DIM), lambda nn, ii: (0, 0)),
        ],
        out_specs=pl.BlockSpec((1, C, ROWS, W_DIM),
                               lambda nn, ii: (nn, 0, jnp.maximum(ii - 1, 0), 0)),
        out_shape=jax.ShapeDtypeStruct((n, C, H, W_DIM), jnp.float32),
        scratch_shapes=[pltpu.VMEM((C, XW), jnp.float32)],
        compiler_params=pltpu.CompilerParams(
            dimension_semantics=("parallel", "arbitrary"),
            vmem_limit_bytes=100 * 1024 * 1024,
        ),
    )(x, w3, bb)
